# baseline (speedup 1.0000x reference)
import functools

import jax
import jax.numpy as jnp
from jax import lax
from jax.experimental import pallas as pl
from jax.experimental.pallas import tpu as pltpu
from jax.experimental.pallas import tpu_sc as plsc

N = 10000
NPAD = 10240
D = 128
E = 320000
G = 10
NPG = 1000
K = 500
KPAD = 512
SLOTS = G * K
NW = 32
S_TILE = 160
SLOTS_PAD = NW * S_TILE
SENTINEL = 1 << 30
CH = 1600
NCH = E // CH
RB = 32


def _tc_body(x_ref, w_ref, b_ref, attended_ref, pos_ref, chosen_ref):
    g = pl.program_id(0)
    xb = x_ref[...]
    w = w_ref[...]
    emb = lax.dot_general(xb, w, (((1,), (1,)), ((), ())),
                          preferred_element_type=jnp.float32) + b_ref[...]
    att = jnp.sum(emb * xb, axis=1, keepdims=True)
    scale = jnp.abs(jnp.tanh(att))
    attended_ref[...] = jnp.maximum(xb * scale + xb, 0.0)

    jr = lax.broadcasted_iota(jnp.int32, (NPG, NPG), 0)
    ir = lax.broadcasted_iota(jnp.int32, (NPG, NPG), 1)
    att_row = jnp.transpose(att)
    a_j = jnp.broadcast_to(att, (NPG, NPG))
    a_i = jnp.broadcast_to(att_row, (NPG, NPG))
    beats = (a_j > a_i) | ((a_j == a_i) & (jr < ir))
    rank_row = jnp.sum(beats.astype(jnp.float32), axis=0, keepdims=True)
    beats_t = (a_i > a_j) | ((a_j == a_i) & (ir < jr))
    rank_col = jnp.sum(beats_t.astype(jnp.float32), axis=1, keepdims=True)

    rr = rank_row.astype(jnp.int32)
    pos = jnp.where(rr < K, g * K + rr, SENTINEL)
    pos_ref[...] = pos.reshape(1, 1, NPG)

    r_lane = lax.broadcasted_iota(jnp.int32, (NPG, KPAD), 1).astype(jnp.float32)
    onehot = (jnp.broadcast_to(rank_col, (NPG, KPAD)) == r_lane).astype(jnp.float32)
    node_iota = lax.broadcasted_iota(jnp.int32, (1, NPG), 1).astype(jnp.float32)
    ch = lax.dot_general(node_iota, onehot, (((1,), (0,)), ((), ())),
                         precision=lax.Precision.HIGHEST,
                         preferred_element_type=jnp.float32)
    chosen_ref[...] = (ch + 0.5).astype(jnp.int32).reshape(1, 1, KPAD) + g * NPG


def _tc_stage(x, W, b, interpret=False):
    return pl.pallas_call(
        _tc_body,
        grid=(G,),
        in_specs=[
            pl.BlockSpec((NPG, D), lambda g: (g, 0)),
            pl.BlockSpec((D, D), lambda g: (0, 0)),
            pl.BlockSpec((1, D), lambda g: (0, 0)),
        ],
        out_specs=[
            pl.BlockSpec((NPG, D), lambda g: (g, 0)),
            pl.BlockSpec((1, 1, NPG), lambda g: (g, 0, 0)),
            pl.BlockSpec((1, 1, KPAD), lambda g: (g, 0, 0)),
        ],
        out_shape=[
            jax.ShapeDtypeStruct((N, D), jnp.float32),
            jax.ShapeDtypeStruct((G, 1, NPG), jnp.int32),
            jax.ShapeDtypeStruct((G, 1, KPAD), jnp.int32),
        ],
        interpret=interpret,
    )(x, W, b.reshape(1, D))


def _sc_body(attended_hbm, pos_hbm, chosen_hbm, src_hbm, dst_hbm, out_hbm,
             spm_att,
             pos_v, nid_v, acc_v, src_v0, dst_v0, src_v1, dst_v1,
             pend_src0, pend_s0, pend_src1, pend_s1, rows_v,
             sem, sem_s0, sem_d0, sem_s1, sem_d1):
    c = lax.axis_index("c")
    s = lax.axis_index("s")
    wid = s * 2 + c
    lo = wid * S_TILE
    iota16 = lax.iota(jnp.int32, 16)
    bufs = ((src_v0, dst_v0, sem_s0, sem_d0), (src_v1, dst_v1, sem_s1, sem_d1))

    def start_chunk(ci, which):
        e0 = jnp.minimum(ci, NCH - 1) * CH
        sv, dv, ss, sd = bufs[which]
        pltpu.async_copy(src_hbm.at[pl.ds(e0, CH)], sv, ss)
        pltpu.async_copy(dst_hbm.at[pl.ds(e0, CH)], dv, sd)

    def wait_chunk(which):
        sv, dv, ss, sd = bufs[which]
        pltpu.make_async_copy(src_hbm.at[pl.ds(0, CH)], sv, ss).wait()
        pltpu.make_async_copy(dst_hbm.at[pl.ds(0, CH)], dv, sd).wait()

    rp = NPAD // 16
    pltpu.sync_copy(attended_hbm.at[pl.ds(s * rp, rp)],
                    spm_att.at[pl.ds(s * rp, rp)])
    pltpu.sync_copy(pos_hbm, pos_v)
    pltpu.sync_copy(chosen_hbm.at[pl.ds(lo, S_TILE)], nid_v)
    plsc.subcore_barrier()

    pltpu.async_copy(spm_att.at[nid_v.at[pl.ds(0, 80)]],
                     acc_v.at[pl.ds(0, 80)], sem).wait()
    pltpu.async_copy(spm_att.at[nid_v.at[pl.ds(80, 80)]],
                     acc_v.at[pl.ds(80, 80)], sem).wait()

    zero16 = jnp.zeros((16,), jnp.int32)

    def _zinit(i, carry):
        pend_src0[pl.ds(i * 16, 16)] = zero16
        pend_src1[pl.ds(i * 16, 16)] = zero16
        return carry

    lax.fori_loop(0, (CH + 16) // 16, _zinit, 0)
    pends = ((pend_src0, pend_s0), (pend_src1, pend_s1))

    def scan_chunk(which):
        sv, dv, _, _ = bufs[which]
        psrc, pslt = pends[which]

        def scan_body(v, np_vec):
            dvec = dv[pl.ds(v * 16, 16)]
            rel = plsc.load_gather(pos_v, [dvec]) - lo
            m = (rel >= 0) & (rel < S_TILE)
            idx = np_vec + plsc.cumsum(m.astype(jnp.int32)) - 1
            svec = sv[pl.ds(v * 16, 16)]
            plsc.store_scatter(psrc, [idx], svec, mask=m)
            plsc.store_scatter(pslt, [idx], rel, mask=m)
            return np_vec + plsc.all_reduce_population_count(m)

        np_vec = lax.fori_loop(0, CH // 16, scan_body,
                               jnp.zeros((16,), jnp.int32))
        return np_vec[0]

    def fire_gather(which, base):
        psrc, _ = pends[which]
        pltpu.async_copy(spm_att.at[psrc.at[pl.ds(base, RB)]], rows_v, sem)

    def wait_gather(which, base):
        psrc, _ = pends[which]
        pltpu.make_async_copy(spm_att.at[psrc.at[pl.ds(base, RB)]],
                              rows_v, sem).wait()

    def run_j(which, base, cnt):
        _, pslt = pends[which]

        def j_body(j, carry3):
            slotv = plsc.load_gather(
                pslt, [jnp.full((16,), base + j, jnp.int32)])
            slot = slotv[0]
            for v8 in range(8):
                sl = pl.ds(v8 * 16, 16)
                val = rows_v[j, sl]
                acc_v[slot, sl] = jnp.maximum(acc_v[slot, sl], val)
            return carry3

        lax.fori_loop(0, cnt, j_body, 0)

    def process_chunk(which, npend):
        wait_gather(which, jnp.int32(0))
        run_j(which, jnp.int32(0), jnp.minimum(RB, npend))
        nb = (npend + (RB - 1)) // RB

        def batch_body(bi, carry2):
            base = bi * RB
            fire_gather(which, base)
            wait_gather(which, base)
            run_j(which, base, jnp.minimum(RB, npend - base))
            return carry2

        lax.fori_loop(1, nb, batch_body, 0)

    start_chunk(jnp.int32(0), 0)
    start_chunk(jnp.int32(1), 1)
    wait_chunk(0)
    np0 = scan_chunk(0)
    fire_gather(0, jnp.int32(0))

    def pair_body(cj, np_a):
        a = cj * 2
        start_chunk(a + 2, 0)
        wait_chunk(1)
        np_b = scan_chunk(1)
        process_chunk(0, np_a)
        fire_gather(1, jnp.int32(0))
        start_chunk(a + 3, 1)
        wait_chunk(0)
        np_c = scan_chunk(0)
        process_chunk(1, np_b)
        fire_gather(0, jnp.int32(0))
        return np_c

    np_last2 = lax.fori_loop(0, NCH // 2 - 1, pair_body, np0)
    wait_chunk(1)
    np_last = scan_chunk(1)
    process_chunk(0, np_last2)
    fire_gather(1, jnp.int32(0))
    process_chunk(1, np_last)
    pltpu.sync_copy(acc_v, out_hbm.at[pl.ds(lo, S_TILE)])


def _sc_stage(attended, pos_flat, chosen_pad, src, dst, interpret=False):
    mesh = plsc.VectorSubcoreMesh(core_axis_name="c", subcore_axis_name="s")
    kern = functools.partial(
        pl.kernel,
        out_type=jax.ShapeDtypeStruct((SLOTS_PAD, D), jnp.float32),
        mesh=mesh,
        compiler_params=pltpu.CompilerParams(needs_layout_passes=False),
        scratch_types=[
            pltpu.VMEM_SHARED((NPAD, D), jnp.float32),
            pltpu.VMEM((N,), jnp.int32),
            pltpu.VMEM((S_TILE,), jnp.int32),
            pltpu.VMEM((S_TILE, D), jnp.float32),
            pltpu.VMEM((CH,), jnp.int32),
            pltpu.VMEM((CH,), jnp.int32),
            pltpu.VMEM((CH,), jnp.int32),
            pltpu.VMEM((CH,), jnp.int32),
            pltpu.VMEM((CH + 16,), jnp.int32),
            pltpu.VMEM((CH + 16,), jnp.int32),
            pltpu.VMEM((CH + 16,), jnp.int32),
            pltpu.VMEM((CH + 16,), jnp.int32),
            pltpu.VMEM((RB, D), jnp.float32),
            pltpu.SemaphoreType.DMA,
            pltpu.SemaphoreType.DMA,
            pltpu.SemaphoreType.DMA,
            pltpu.SemaphoreType.DMA,
            pltpu.SemaphoreType.DMA,
        ],
        interpret=interpret,
    )(_sc_body)
    return kern(attended, pos_flat, chosen_pad, src, dst)


def kernel(x, edge_index, num_graphs, W, b):
    attended, pos3, chosen3 = _tc_stage(x, W, b)
    pos_flat = pos3.reshape(N)
    chosen = chosen3.reshape(G, KPAD)[:, :K].reshape(SLOTS)
    chosen_pad = jnp.concatenate(
        [chosen, jnp.zeros((SLOTS_PAD - SLOTS,), jnp.int32)])
    att_pad = jnp.concatenate(
        [attended, jnp.zeros((NPAD - N, D), jnp.float32)])
    out_pad = _sc_stage(att_pad, pos_flat, chosen_pad,
                        edge_index[0], edge_index[1])
    return (out_pad[:SLOTS], chosen)


# speedup vs baseline: 1.1986x; 1.1986x over previous
import functools

import jax
import jax.numpy as jnp
from jax import lax
from jax.experimental import pallas as pl
from jax.experimental.pallas import tpu as pltpu
from jax.experimental.pallas import tpu_sc as plsc

N = 10000
NPAD = 10240
D = 128
E = 320000
G = 10
NPG = 1000
K = 500
KPAD = 512
SLOTS = G * K
NW = 32
S_TILE = 160
SLOTS_PAD = NW * S_TILE
SENTINEL = 1 << 30
CH = 1600
NCH = E // CH
RB = 32


def _tc_body(x_ref, w_ref, b_ref, attended_ref, pos_ref, chosen_ref):
    g = pl.program_id(0)
    xb = x_ref[...]
    w = w_ref[...]
    emb = lax.dot_general(xb, w, (((1,), (1,)), ((), ())),
                          preferred_element_type=jnp.float32) + b_ref[...]
    att = jnp.sum(emb * xb, axis=1, keepdims=True)
    scale = jnp.abs(jnp.tanh(att))
    attended_ref[...] = jnp.maximum(xb * scale + xb, 0.0)

    jr = lax.broadcasted_iota(jnp.int32, (NPG, NPG), 0)
    ir = lax.broadcasted_iota(jnp.int32, (NPG, NPG), 1)
    att_row = jnp.transpose(att)
    a_j = jnp.broadcast_to(att, (NPG, NPG))
    a_i = jnp.broadcast_to(att_row, (NPG, NPG))
    beats = (a_j > a_i) | ((a_j == a_i) & (jr < ir))
    rank_row = jnp.sum(beats.astype(jnp.float32), axis=0, keepdims=True)
    beats_t = (a_i > a_j) | ((a_j == a_i) & (ir < jr))
    rank_col = jnp.sum(beats_t.astype(jnp.float32), axis=1, keepdims=True)

    rr = rank_row.astype(jnp.int32)
    pos = jnp.where(rr < K, g * K + rr, SENTINEL)
    pos_ref[...] = pos.reshape(1, 1, NPG)

    r_lane = lax.broadcasted_iota(jnp.int32, (NPG, KPAD), 1).astype(jnp.float32)
    onehot = (jnp.broadcast_to(rank_col, (NPG, KPAD)) == r_lane).astype(jnp.float32)
    node_iota = lax.broadcasted_iota(jnp.int32, (1, NPG), 1).astype(jnp.float32)
    ch = lax.dot_general(node_iota, onehot, (((1,), (0,)), ((), ())),
                         precision=lax.Precision.HIGHEST,
                         preferred_element_type=jnp.float32)
    chosen_ref[...] = (ch + 0.5).astype(jnp.int32).reshape(1, 1, KPAD) + g * NPG


def _tc_stage(x, W, b, interpret=False):
    return pl.pallas_call(
        _tc_body,
        grid=(G,),
        in_specs=[
            pl.BlockSpec((NPG, D), lambda g: (g, 0)),
            pl.BlockSpec((D, D), lambda g: (0, 0)),
            pl.BlockSpec((1, D), lambda g: (0, 0)),
        ],
        out_specs=[
            pl.BlockSpec((NPG, D), lambda g: (g, 0)),
            pl.BlockSpec((1, 1, NPG), lambda g: (g, 0, 0)),
            pl.BlockSpec((1, 1, KPAD), lambda g: (g, 0, 0)),
        ],
        out_shape=[
            jax.ShapeDtypeStruct((N, D), jnp.float32),
            jax.ShapeDtypeStruct((G, 1, NPG), jnp.int32),
            jax.ShapeDtypeStruct((G, 1, KPAD), jnp.int32),
        ],
        interpret=interpret,
    )(x, W, b.reshape(1, D))


def _sc_body(attended_hbm, pos_hbm, chosen_hbm, src_hbm, dst_hbm, out_hbm,
             spm_att,
             pos_v, nid_v, acc_v, src_v0, dst_v0, src_v1, dst_v1,
             pend_src0, pend_s0, pend_src1, pend_s1, rows_v,
             sem, sem_s0, sem_d0, sem_s1, sem_d1):
    c = lax.axis_index("c")
    s = lax.axis_index("s")
    wid = s * 2 + c
    lo = wid * S_TILE
    iota16 = lax.iota(jnp.int32, 16)
    bufs = ((src_v0, dst_v0, sem_s0, sem_d0), (src_v1, dst_v1, sem_s1, sem_d1))

    def start_chunk(ci, which):
        e0 = jnp.minimum(ci, NCH - 1) * CH
        sv, dv, ss, sd = bufs[which]
        pltpu.async_copy(src_hbm.at[pl.ds(e0, CH)], sv, ss)
        pltpu.async_copy(dst_hbm.at[pl.ds(e0, CH)], dv, sd)

    def wait_chunk(which):
        sv, dv, ss, sd = bufs[which]
        pltpu.make_async_copy(src_hbm.at[pl.ds(0, CH)], sv, ss).wait()
        pltpu.make_async_copy(dst_hbm.at[pl.ds(0, CH)], dv, sd).wait()

    rp = NPAD // 16
    pltpu.sync_copy(attended_hbm.at[pl.ds(s * rp, rp)],
                    spm_att.at[pl.ds(s * rp, rp)])
    pltpu.sync_copy(pos_hbm, pos_v)
    pltpu.sync_copy(chosen_hbm.at[pl.ds(lo, S_TILE)], nid_v)
    plsc.subcore_barrier()

    pltpu.async_copy(spm_att.at[nid_v.at[pl.ds(0, 80)]],
                     acc_v.at[pl.ds(0, 80)], sem).wait()
    pltpu.async_copy(spm_att.at[nid_v.at[pl.ds(80, 80)]],
                     acc_v.at[pl.ds(80, 80)], sem).wait()

    zero16 = jnp.zeros((16,), jnp.int32)

    def _zinit(i, carry):
        pend_src0[pl.ds(i * 16, 16)] = zero16
        pend_src1[pl.ds(i * 16, 16)] = zero16
        return carry

    lax.fori_loop(0, (CH + 16) // 16, _zinit, 0)
    pends = ((pend_src0, pend_s0), (pend_src1, pend_s1))

    def scan_chunk(which):
        sv, dv, _, _ = bufs[which]
        psrc, pslt = pends[which]

        def scan_body(v, np_cnt):
            dvec = dv[pl.ds(v * 16, 16)]
            rel = plsc.load_gather(pos_v, [dvec]) - lo
            m = (rel >= 0) & (rel < S_TILE)
            svec = sv[pl.ds(v * 16, 16)]
            plsc.store_compressed(psrc.at[pl.ds(np_cnt, 16)], svec, mask=m)
            plsc.store_compressed(pslt.at[pl.ds(np_cnt, 16)], rel, mask=m)
            return np_cnt + plsc.all_reduce_population_count(m)[0]

        return lax.fori_loop(0, CH // 16, scan_body, jnp.int32(0))

    def fire_gather(which, base):
        psrc, _ = pends[which]
        pltpu.async_copy(spm_att.at[psrc.at[pl.ds(base, RB)]], rows_v, sem)

    def wait_gather(which, base):
        psrc, _ = pends[which]
        pltpu.make_async_copy(spm_att.at[psrc.at[pl.ds(base, RB)]],
                              rows_v, sem).wait()

    def run_j(which, base, cnt):
        _, pslt = pends[which]

        def one(j):
            slotv = plsc.load_gather(
                pslt, [jnp.full((16,), base + j, jnp.int32)])
            slot = slotv[0]
            for v8 in range(8):
                sl = pl.ds(v8 * 16, 16)
                val = rows_v[j, sl]
                acc_v[slot, sl] = jnp.maximum(acc_v[slot, sl], val)

        def pair_j(jp, carry3):
            j0 = jp * 2
            j1 = j0 + 1
            slotv0 = plsc.load_gather(
                pslt, [jnp.full((16,), base + j0, jnp.int32)])
            slotv1 = plsc.load_gather(
                pslt, [jnp.full((16,), base + j1, jnp.int32)])
            s0 = slotv0[0]
            s1 = slotv1[0]
            for v8 in range(8):
                sl = pl.ds(v8 * 16, 16)
                v0 = rows_v[j0, sl]
                v1 = rows_v[j1, sl]
                acc_v[s0, sl] = jnp.maximum(acc_v[s0, sl], v0)
                acc_v[s1, sl] = jnp.maximum(acc_v[s1, sl], v1)
            return carry3

        def tail_j(j, carry3):
            one(j)
            return carry3

        lax.fori_loop(0, cnt // 2, pair_j, 0)
        lax.fori_loop(cnt & ~1, cnt, tail_j, 0)

    def process_chunk(which, npend):
        wait_gather(which, jnp.int32(0))
        run_j(which, jnp.int32(0), jnp.minimum(RB, npend))
        nb = (npend + (RB - 1)) // RB

        def batch_body(bi, carry2):
            base = bi * RB
            fire_gather(which, base)
            wait_gather(which, base)
            run_j(which, base, jnp.minimum(RB, npend - base))
            return carry2

        lax.fori_loop(1, nb, batch_body, 0)

    start_chunk(jnp.int32(0), 0)
    start_chunk(jnp.int32(1), 1)
    wait_chunk(0)
    np0 = scan_chunk(0)
    fire_gather(0, jnp.int32(0))

    def pair_body(cj, np_a):
        a = cj * 2
        start_chunk(a + 2, 0)
        wait_chunk(1)
        np_b = scan_chunk(1)
        process_chunk(0, np_a)
        fire_gather(1, jnp.int32(0))
        start_chunk(a + 3, 1)
        wait_chunk(0)
        np_c = scan_chunk(0)
        process_chunk(1, np_b)
        fire_gather(0, jnp.int32(0))
        return np_c

    np_last2 = lax.fori_loop(0, NCH // 2 - 1, pair_body, np0)
    wait_chunk(1)
    np_last = scan_chunk(1)
    process_chunk(0, np_last2)
    fire_gather(1, jnp.int32(0))
    process_chunk(1, np_last)
    pltpu.sync_copy(acc_v, out_hbm.at[pl.ds(lo, S_TILE)])


def _sc_stage(attended, pos_flat, chosen_pad, src, dst, interpret=False):
    mesh = plsc.VectorSubcoreMesh(core_axis_name="c", subcore_axis_name="s")
    kern = functools.partial(
        pl.kernel,
        out_type=jax.ShapeDtypeStruct((SLOTS_PAD, D), jnp.float32),
        mesh=mesh,
        compiler_params=pltpu.CompilerParams(needs_layout_passes=False),
        scratch_types=[
            pltpu.VMEM_SHARED((NPAD, D), jnp.float32),
            pltpu.VMEM((N,), jnp.int32),
            pltpu.VMEM((S_TILE,), jnp.int32),
            pltpu.VMEM((S_TILE, D), jnp.float32),
            pltpu.VMEM((CH,), jnp.int32),
            pltpu.VMEM((CH,), jnp.int32),
            pltpu.VMEM((CH,), jnp.int32),
            pltpu.VMEM((CH,), jnp.int32),
            pltpu.VMEM((CH + 16,), jnp.int32),
            pltpu.VMEM((CH + 16,), jnp.int32),
            pltpu.VMEM((CH + 16,), jnp.int32),
            pltpu.VMEM((CH + 16,), jnp.int32),
            pltpu.VMEM((RB, D), jnp.float32),
            pltpu.SemaphoreType.DMA,
            pltpu.SemaphoreType.DMA,
            pltpu.SemaphoreType.DMA,
            pltpu.SemaphoreType.DMA,
            pltpu.SemaphoreType.DMA,
        ],
        interpret=interpret,
    )(_sc_body)
    return kern(attended, pos_flat, chosen_pad, src, dst)


def kernel(x, edge_index, num_graphs, W, b):
    attended, pos3, chosen3 = _tc_stage(x, W, b)
    pos_flat = pos3.reshape(N)
    chosen = chosen3.reshape(G, KPAD)[:, :K].reshape(SLOTS)
    chosen_pad = jnp.concatenate(
        [chosen, jnp.zeros((SLOTS_PAD - SLOTS,), jnp.int32)])
    att_pad = jnp.concatenate(
        [attended, jnp.zeros((NPAD - N, D), jnp.float32)])
    out_pad = _sc_stage(att_pad, pos_flat, chosen_pad,
                        edge_index[0], edge_index[1])
    return (out_pad[:SLOTS], chosen)


# speedup vs baseline: 1.2065x; 1.0065x over previous
import functools

import jax
import jax.numpy as jnp
from jax import lax
from jax.experimental import pallas as pl
from jax.experimental.pallas import tpu as pltpu
from jax.experimental.pallas import tpu_sc as plsc

N = 10000
NPAD = 10240
D = 128
E = 320000
G = 10
NPG = 1000
K = 500
KPAD = 512
SLOTS = G * K
NW = 32
S_TILE = 160
SLOTS_PAD = NW * S_TILE
SENTINEL = 1 << 30
CH = 1600
NCH = E // CH
RB = 32


def _tc_body(x_ref, w_ref, b_ref, attended_ref, pos_ref, chosen_ref):
    g = pl.program_id(0)
    xb = x_ref[...]
    w = w_ref[...]
    emb = lax.dot_general(xb, w, (((1,), (1,)), ((), ())),
                          preferred_element_type=jnp.float32) + b_ref[...]
    att = jnp.sum(emb * xb, axis=1, keepdims=True)
    scale = jnp.abs(jnp.tanh(att))
    attended_ref[...] = jnp.maximum(xb * scale + xb, 0.0)

    jr = lax.broadcasted_iota(jnp.int32, (NPG, NPG), 0)
    ir = lax.broadcasted_iota(jnp.int32, (NPG, NPG), 1)
    att_row = jnp.transpose(att)
    a_j = jnp.broadcast_to(att, (NPG, NPG))
    a_i = jnp.broadcast_to(att_row, (NPG, NPG))
    beats = (a_j > a_i) | ((a_j == a_i) & (jr < ir))
    rank_row = jnp.sum(beats.astype(jnp.float32), axis=0, keepdims=True)
    beats_t = (a_i > a_j) | ((a_j == a_i) & (ir < jr))
    rank_col = jnp.sum(beats_t.astype(jnp.float32), axis=1, keepdims=True)

    rr = rank_row.astype(jnp.int32)
    pos = jnp.where(rr < K, g * K + rr, SENTINEL)
    pos_ref[...] = pos.reshape(1, 1, NPG)

    r_lane = lax.broadcasted_iota(jnp.int32, (NPG, KPAD), 1).astype(jnp.float32)
    onehot = (jnp.broadcast_to(rank_col, (NPG, KPAD)) == r_lane).astype(jnp.float32)
    node_iota = lax.broadcasted_iota(jnp.int32, (1, NPG), 1).astype(jnp.float32)
    ch = lax.dot_general(node_iota, onehot, (((1,), (0,)), ((), ())),
                         precision=lax.Precision.HIGHEST,
                         preferred_element_type=jnp.float32)
    chosen_ref[...] = (ch + 0.5).astype(jnp.int32).reshape(1, 1, KPAD) + g * NPG


def _tc_stage(x, W, b, interpret=False):
    return pl.pallas_call(
        _tc_body,
        grid=(G,),
        in_specs=[
            pl.BlockSpec((NPG, D), lambda g: (g, 0)),
            pl.BlockSpec((D, D), lambda g: (0, 0)),
            pl.BlockSpec((1, D), lambda g: (0, 0)),
        ],
        out_specs=[
            pl.BlockSpec((NPG, D), lambda g: (g, 0)),
            pl.BlockSpec((1, 1, NPG), lambda g: (g, 0, 0)),
            pl.BlockSpec((1, 1, KPAD), lambda g: (g, 0, 0)),
        ],
        out_shape=[
            jax.ShapeDtypeStruct((N, D), jnp.float32),
            jax.ShapeDtypeStruct((G, 1, NPG), jnp.int32),
            jax.ShapeDtypeStruct((G, 1, KPAD), jnp.int32),
        ],
        interpret=interpret,
    )(x, W, b.reshape(1, D))


def _sc_body(attended_hbm, pos_hbm, chosen_hbm, src_hbm, dst_hbm, out_hbm,
             spm_att,
             pos_v, nid_v, acc_v, src_v0, dst_v0, src_v1, dst_v1,
             pend_src0, pend_s0, pend_src1, pend_s1, rows_v,
             sem, sem_s0, sem_d0, sem_s1, sem_d1):
    c = lax.axis_index("c")
    s = lax.axis_index("s")
    wid = s * 2 + c
    lo = wid * S_TILE
    iota16 = lax.iota(jnp.int32, 16)
    bufs = ((src_v0, dst_v0, sem_s0, sem_d0), (src_v1, dst_v1, sem_s1, sem_d1))

    def start_chunk(ci, which):
        e0 = jnp.minimum(ci, NCH - 1) * CH
        sv, dv, ss, sd = bufs[which]
        pltpu.async_copy(src_hbm.at[pl.ds(e0, CH)], sv, ss)
        pltpu.async_copy(dst_hbm.at[pl.ds(e0, CH)], dv, sd)

    def wait_chunk(which):
        sv, dv, ss, sd = bufs[which]
        pltpu.make_async_copy(src_hbm.at[pl.ds(0, CH)], sv, ss).wait()
        pltpu.make_async_copy(dst_hbm.at[pl.ds(0, CH)], dv, sd).wait()

    rp = NPAD // 16
    pltpu.sync_copy(attended_hbm.at[pl.ds(s * rp, rp)],
                    spm_att.at[pl.ds(s * rp, rp)])
    pltpu.sync_copy(pos_hbm, pos_v)
    pltpu.sync_copy(chosen_hbm.at[pl.ds(lo, S_TILE)], nid_v)
    plsc.subcore_barrier()

    pltpu.async_copy(spm_att.at[nid_v.at[pl.ds(0, 80)]],
                     acc_v.at[pl.ds(0, 80)], sem).wait()
    pltpu.async_copy(spm_att.at[nid_v.at[pl.ds(80, 80)]],
                     acc_v.at[pl.ds(80, 80)], sem).wait()

    zero16 = jnp.zeros((16,), jnp.int32)

    def _zinit(i, carry):
        pend_src0[pl.ds(i * 16, 16)] = zero16
        pend_src1[pl.ds(i * 16, 16)] = zero16
        return carry

    lax.fori_loop(0, (CH + 16) // 16, _zinit, 0)
    pends = ((pend_src0, pend_s0), (pend_src1, pend_s1))

    def scan_chunk(which):
        sv, dv, _, _ = bufs[which]
        psrc, pslt = pends[which]

        def scan_body(v, np_cnt):
            dvec = dv[pl.ds(v * 16, 16)]
            rel = plsc.load_gather(pos_v, [dvec]) - lo
            m = (rel >= 0) & (rel < S_TILE)
            svec = sv[pl.ds(v * 16, 16)]
            plsc.store_compressed(psrc.at[pl.ds(np_cnt, 16)], svec, mask=m)
            plsc.store_compressed(pslt.at[pl.ds(np_cnt, 16)], rel, mask=m)
            return np_cnt + plsc.all_reduce_population_count(m)[0]

        return lax.fori_loop(0, CH // 16, scan_body, jnp.int32(0))

    def fire_gather(which, base):
        psrc, _ = pends[which]
        pltpu.async_copy(spm_att.at[psrc.at[pl.ds(base, RB)]], rows_v, sem)

    def wait_gather(which, base):
        psrc, _ = pends[which]
        pltpu.make_async_copy(spm_att.at[psrc.at[pl.ds(base, RB)]],
                              rows_v, sem).wait()

    def run_j(which, base, cnt):
        _, pslt = pends[which]

        def one(j):
            slotv = plsc.load_gather(
                pslt, [jnp.full((16,), base + j, jnp.int32)])
            slot = slotv[0]
            for v8 in range(8):
                sl = pl.ds(v8 * 16, 16)
                val = rows_v[j, sl]
                acc_v[slot, sl] = jnp.maximum(acc_v[slot, sl], val)

        def quad_j(jp, carry3):
            js = [jp * 4 + t for t in range(4)]
            ss = []
            for j in js:
                slotv = plsc.load_gather(
                    pslt, [jnp.full((16,), base + j, jnp.int32)])
                ss.append(slotv[0])
            for v8 in range(8):
                sl = pl.ds(v8 * 16, 16)
                for j, sj in zip(js, ss):
                    acc_v[sj, sl] = jnp.maximum(acc_v[sj, sl], rows_v[j, sl])
            return carry3

        def tail_j(j, carry3):
            one(j)
            return carry3

        lax.fori_loop(0, cnt // 4, quad_j, 0)
        lax.fori_loop(cnt & ~3, cnt, tail_j, 0)

    def process_chunk(which, npend):
        wait_gather(which, jnp.int32(0))
        run_j(which, jnp.int32(0), jnp.minimum(RB, npend))
        nb = (npend + (RB - 1)) // RB

        def batch_body(bi, carry2):
            base = bi * RB
            fire_gather(which, base)
            wait_gather(which, base)
            run_j(which, base, jnp.minimum(RB, npend - base))
            return carry2

        lax.fori_loop(1, nb, batch_body, 0)

    start_chunk(jnp.int32(0), 0)
    start_chunk(jnp.int32(1), 1)
    wait_chunk(0)
    np0 = scan_chunk(0)
    fire_gather(0, jnp.int32(0))

    def pair_body(cj, np_a):
        a = cj * 2
        start_chunk(a + 2, 0)
        wait_chunk(1)
        np_b = scan_chunk(1)
        process_chunk(0, np_a)
        fire_gather(1, jnp.int32(0))
        start_chunk(a + 3, 1)
        wait_chunk(0)
        np_c = scan_chunk(0)
        process_chunk(1, np_b)
        fire_gather(0, jnp.int32(0))
        return np_c

    np_last2 = lax.fori_loop(0, NCH // 2 - 1, pair_body, np0)
    wait_chunk(1)
    np_last = scan_chunk(1)
    process_chunk(0, np_last2)
    fire_gather(1, jnp.int32(0))
    process_chunk(1, np_last)
    pltpu.sync_copy(acc_v, out_hbm.at[pl.ds(lo, S_TILE)])


def _sc_stage(attended, pos_flat, chosen_pad, src, dst, interpret=False):
    mesh = plsc.VectorSubcoreMesh(core_axis_name="c", subcore_axis_name="s")
    kern = functools.partial(
        pl.kernel,
        out_type=jax.ShapeDtypeStruct((SLOTS_PAD, D), jnp.float32),
        mesh=mesh,
        compiler_params=pltpu.CompilerParams(needs_layout_passes=False),
        scratch_types=[
            pltpu.VMEM_SHARED((NPAD, D), jnp.float32),
            pltpu.VMEM((N,), jnp.int32),
            pltpu.VMEM((S_TILE,), jnp.int32),
            pltpu.VMEM((S_TILE, D), jnp.float32),
            pltpu.VMEM((CH,), jnp.int32),
            pltpu.VMEM((CH,), jnp.int32),
            pltpu.VMEM((CH,), jnp.int32),
            pltpu.VMEM((CH,), jnp.int32),
            pltpu.VMEM((CH + 16,), jnp.int32),
            pltpu.VMEM((CH + 16,), jnp.int32),
            pltpu.VMEM((CH + 16,), jnp.int32),
            pltpu.VMEM((CH + 16,), jnp.int32),
            pltpu.VMEM((RB, D), jnp.float32),
            pltpu.SemaphoreType.DMA,
            pltpu.SemaphoreType.DMA,
            pltpu.SemaphoreType.DMA,
            pltpu.SemaphoreType.DMA,
            pltpu.SemaphoreType.DMA,
        ],
        interpret=interpret,
    )(_sc_body)
    return kern(attended, pos_flat, chosen_pad, src, dst)


def kernel(x, edge_index, num_graphs, W, b):
    attended, pos3, chosen3 = _tc_stage(x, W, b)
    pos_flat = pos3.reshape(N)
    chosen = chosen3.reshape(G, KPAD)[:, :K].reshape(SLOTS)
    chosen_pad = jnp.concatenate(
        [chosen, jnp.zeros((SLOTS_PAD - SLOTS,), jnp.int32)])
    att_pad = jnp.concatenate(
        [attended, jnp.zeros((NPAD - N, D), jnp.float32)])
    out_pad = _sc_stage(att_pad, pos_flat, chosen_pad,
                        edge_index[0], edge_index[1])
    return (out_pad[:SLOTS], chosen)
